# Initial kernel scaffold; baseline (speedup 1.0000x reference)
import jax
import jax.numpy as jnp
from jax.experimental import pallas as pl


def kernel(input, target, batch):
    raise NotImplementedError("write your pallas kernel here")


# speedup vs baseline: 2.7017x; 2.7017x over previous
import dataclasses
import functools

import jax
import jax.numpy as jnp
from jax import lax
from jax.experimental import pallas as pl
from jax.experimental.pallas import tpu as pltpu
from jax.experimental.pallas import tpu_sc as plsc

N = 32768
NSEG = 16
CAP = 4096
NB = 1024
LN = 16
SENT = (1 << 30) - 1
EPS = 1e-12


def _pre_body(tgt_ref, inp_ref, bat_ref, key_ref, val_ref, starts_ref,
              counts_ref):
  tb = lax.bitcast_convert_type(tgt_ref[...], jnp.int32)
  key_ref[...] = jnp.int32(0x3F7FFFFF) - tb
  val_ref[...] = jnp.exp(inp_ref[...])
  b = bat_ref[...]
  run = jnp.int32(0)
  for s in range(NSEG):
    cnt = jnp.sum((b == s).astype(jnp.int32))
    counts_ref[0, s] = cnt
    starts_ref[0, s] = run
    run = run + cnt


_pre = pl.pallas_call(
    _pre_body,
    out_shape=(
        jax.ShapeDtypeStruct((8, N // 8), jnp.int32),
        jax.ShapeDtypeStruct((8, N // 8), jnp.float32),
        jax.ShapeDtypeStruct((1, NSEG), jnp.int32),
        jax.ShapeDtypeStruct((1, NSEG), jnp.int32),
    ),
    out_specs=(
        pl.BlockSpec(memory_space=pltpu.VMEM),
        pl.BlockSpec(memory_space=pltpu.VMEM),
        pl.BlockSpec(memory_space=pltpu.SMEM),
        pl.BlockSpec(memory_space=pltpu.SMEM),
    ),
)


def _sc_compiler_params():
  cp = pltpu.CompilerParams()
  if "needs_layout_passes" in pltpu.CompilerParams.__dataclass_fields__:
    cp = dataclasses.replace(cp, needs_layout_passes=False)
  return cp


_sc_mesh = plsc.VectorSubcoreMesh(core_axis_name="c", subcore_axis_name="s")


@functools.partial(
    pl.kernel,
    out_type=jax.ShapeDtypeStruct((NSEG * CAP,), jnp.float32),
    mesh=_sc_mesh,
    scratch_types=[
        pltpu.VMEM((CAP,), jnp.int32),
        pltpu.VMEM((CAP,), jnp.float32),
        pltpu.VMEM((CAP,), jnp.int32),
        pltpu.VMEM((CAP,), jnp.float32),
        pltpu.VMEM((CAP,), jnp.int32),
        pltpu.VMEM((NB,), jnp.int32),
        pltpu.VMEM((NSEG,), jnp.int32),
        pltpu.VMEM((CAP,), jnp.float32),
    ],
    compiler_params=_sc_compiler_params(),
)
def _sc_segsort(key_hbm, val_hbm, bat_hbm, starts_hbm, out_hbm,
                key_a, val_a, key_b, val_b, bwin, bins, st, outs):
  c = lax.axis_index("c")
  s = lax.axis_index("s")

  @pl.when(c == 0)
  def _worker():
    pltpu.sync_copy(starts_hbm, st)
    stv = st[...]
    li = lax.iota(jnp.int32, 16)
    my_start = jnp.sum(jnp.where(li == s, stv, 0))
    nxt = jnp.sum(jnp.where(li == s + 1, stv, 0))
    seg_end = jnp.where(s == NSEG - 1, jnp.int32(N), nxt)
    base = pl.multiple_of(
        jnp.minimum(my_start & jnp.int32(~7), jnp.int32(N - CAP)), 8)
    o_end = seg_end - base
    nvr = (o_end + LN - 1) >> 4

    pltpu.sync_copy(key_hbm.at[pl.ds(base, CAP)], key_a)
    pltpu.sync_copy(val_hbm.at[pl.ds(base, CAP)], val_a)
    pltpu.sync_copy(bat_hbm.at[pl.ds(base, CAP)], bwin)

    @pl.loop(0, nvr)
    def _mask(i):
      sl = pl.ds(i * LN, LN)
      mine = bwin[sl] == s
      key_a[sl] = jnp.where(mine, key_a[sl], jnp.int32(SENT))
      val_a[sl] = jnp.where(mine, val_a[sl], 0.0)

    for p, (ksrc, vsrc, kdst, vdst) in enumerate((
        (key_a, val_a, key_b, val_b),
        (key_b, val_b, key_a, val_a),
        (key_a, val_a, key_b, val_b),
    )):
      shift = jnp.int32(p * 10)

      @pl.loop(0, NB // LN)
      def _zero(j):
        bins[pl.ds(j * LN, LN)] = jnp.zeros((LN,), jnp.int32)

      @pl.loop(0, nvr)
      def _hist(i):
        k = ksrc[pl.ds(i * LN, LN)]
        d = lax.shift_right_logical(k, shift) & jnp.int32(NB - 1)
        cnt, lastm = plsc.scan_count(d)
        plsc.addupdate_scatter(bins, [d], cnt, mask=lastm)

      def _scan(j, carry):
        sl = pl.ds(j * LN, LN)
        v = bins[sl]
        inc = plsc.cumsum(v)
        bins[sl] = inc - v + carry
        return carry + jnp.sum(v)

      pl.loop(0, NB // LN, init_carry=jnp.int32(0))(_scan)

      @pl.loop(0, nvr)
      def _permute(i):
        sl = pl.ds(i * LN, LN)
        k = ksrc[sl]
        x = vsrc[sl]
        d = lax.shift_right_logical(k, shift) & jnp.int32(NB - 1)
        cnt, lastm = plsc.scan_count(d)
        pos = plsc.load_gather(bins, [d]) + cnt - 1
        plsc.store_scatter(kdst, [pos], k)
        plsc.store_scatter(vdst, [pos], x)
        plsc.addupdate_scatter(bins, [d], cnt, mask=lastm)

    def _suffix(i, carry):
      sl = pl.ds((nvr - 1 - i) * LN, LN)
      v = val_b[sl]
      rs = plsc.cumsum(lax.rev(v, (0,)))
      outs[sl] = lax.rev(rs, (0,)) + carry
      return carry + jnp.sum(v)

    pl.loop(0, nvr, init_carry=jnp.float32(0.0))(_suffix)

    pltpu.sync_copy(outs, out_hbm.at[pl.ds(pl.multiple_of(s * CAP, 8), CAP)])


def _post_body(suf_ref, counts_ref, inp_ref, out_ref):
  col = lax.broadcasted_iota(jnp.int32, (1, CAP), 1)
  acc = jnp.zeros((1, CAP), jnp.float32)
  for s in range(NSEG):
    row = suf_ref[pl.ds(s, 1), :]
    m = counts_ref[0, s]
    acc = acc + jnp.where(col < m, jnp.log(row + EPS), 0.0)
  total = jnp.sum(acc) - jnp.sum(inp_ref[...])
  out_ref[0, 0] = total / NSEG


_post = pl.pallas_call(
    _post_body,
    out_shape=jax.ShapeDtypeStruct((1, 1), jnp.float32),
    in_specs=(
        pl.BlockSpec(memory_space=pltpu.VMEM),
        pl.BlockSpec(memory_space=pltpu.SMEM),
        pl.BlockSpec(memory_space=pltpu.VMEM),
    ),
    out_specs=pl.BlockSpec(memory_space=pltpu.SMEM),
)


def kernel(input, target, batch):
  inp2 = input.reshape(8, N // 8)
  tgt2 = target.reshape(8, N // 8)
  bat2 = batch.astype(jnp.int32).reshape(8, N // 8)
  keys2, vals2, starts, counts = _pre(tgt2, inp2, bat2)
  suffix = _sc_segsort(
      keys2.reshape(N),
      vals2.reshape(N),
      batch.astype(jnp.int32),
      starts.reshape(NSEG),
  )
  out = _post(suffix.reshape(NSEG, CAP), counts, inp2)
  return out.reshape(())


# speedup vs baseline: 3.3086x; 1.2246x over previous
import dataclasses
import functools

import jax
import jax.numpy as jnp
from jax import lax
from jax.experimental import pallas as pl
from jax.experimental.pallas import tpu as pltpu
from jax.experimental.pallas import tpu_sc as plsc

N = 32768
NSEG = 16
CAP = 4096
NB = 1024
LN = 16
SENT = (1 << 30) - 1
EPS = 1e-12


def _pre_body(tgt_ref, inp_ref, bat_ref, key_ref, val_ref, starts_ref,
              counts_ref):
  tb = lax.bitcast_convert_type(tgt_ref[...], jnp.int32)
  key_ref[...] = jnp.int32(0x3F7FFFFF) - tb
  val_ref[...] = jnp.exp(inp_ref[...])
  b = bat_ref[...]
  run = jnp.int32(0)
  for s in range(NSEG):
    cnt = jnp.sum((b == s).astype(jnp.int32))
    counts_ref[0, s] = cnt
    starts_ref[0, s] = run
    run = run + cnt


_pre = pl.pallas_call(
    _pre_body,
    out_shape=(
        jax.ShapeDtypeStruct((8, N // 8), jnp.int32),
        jax.ShapeDtypeStruct((8, N // 8), jnp.float32),
        jax.ShapeDtypeStruct((1, NSEG), jnp.int32),
        jax.ShapeDtypeStruct((1, NSEG), jnp.int32),
    ),
    out_specs=(
        pl.BlockSpec(memory_space=pltpu.VMEM),
        pl.BlockSpec(memory_space=pltpu.VMEM),
        pl.BlockSpec(memory_space=pltpu.SMEM),
        pl.BlockSpec(memory_space=pltpu.SMEM),
    ),
)


def _sc_compiler_params():
  cp = pltpu.CompilerParams()
  if "needs_layout_passes" in pltpu.CompilerParams.__dataclass_fields__:
    cp = dataclasses.replace(cp, needs_layout_passes=False)
  return cp


_sc_mesh = plsc.VectorSubcoreMesh(core_axis_name="c", subcore_axis_name="s")


@functools.partial(
    pl.kernel,
    out_type=jax.ShapeDtypeStruct((NSEG * CAP,), jnp.float32),
    mesh=_sc_mesh,
    scratch_types=[
        pltpu.VMEM((CAP,), jnp.int32),
        pltpu.VMEM((CAP,), jnp.float32),
        pltpu.VMEM((CAP,), jnp.int32),
        pltpu.VMEM((CAP,), jnp.float32),
        pltpu.VMEM((CAP,), jnp.int32),
        pltpu.VMEM((NB,), jnp.int32),
        pltpu.VMEM((NB,), jnp.int32),
        pltpu.VMEM((NB,), jnp.int32),
        pltpu.VMEM((NSEG,), jnp.int32),
        pltpu.VMEM((CAP,), jnp.float32),
    ],
    compiler_params=_sc_compiler_params(),
)
def _sc_segsort(key_hbm, val_hbm, bat_hbm, starts_hbm, out_hbm,
                key_a, val_a, key_b, val_b, bwin, bins0, bins1, bins2,
                st, outs):
  c = lax.axis_index("c")
  s = lax.axis_index("s")

  @pl.when(c == 0)
  def _worker():
    pltpu.sync_copy(starts_hbm, st)
    stv = st[...]
    li = lax.iota(jnp.int32, 16)
    my_start = jnp.sum(jnp.where(li == s, stv, 0))
    nxt = jnp.sum(jnp.where(li == s + 1, stv, 0))
    seg_end = jnp.where(s == NSEG - 1, jnp.int32(N), nxt)
    base = pl.multiple_of(
        jnp.minimum(my_start & jnp.int32(~7), jnp.int32(N - CAP)), 8)
    o_end = seg_end - base
    nvr = (o_end + LN - 1) >> 4

    pltpu.sync_copy(key_hbm.at[pl.ds(base, CAP)], key_a)
    pltpu.sync_copy(val_hbm.at[pl.ds(base, CAP)], val_a)
    pltpu.sync_copy(bat_hbm.at[pl.ds(base, CAP)], bwin)

    all_bins = (bins0, bins1, bins2)

    @pl.loop(0, NB // LN)
    def _zero(j):
      z = jnp.zeros((LN,), jnp.int32)
      for b in all_bins:
        b[pl.ds(j * LN, LN)] = z

    @pl.loop(0, nvr)
    def _mask_hist(i):
      sl = pl.ds(i * LN, LN)
      mine = bwin[sl] == s
      k = jnp.where(mine, key_a[sl], jnp.int32(SENT))
      key_a[sl] = k
      val_a[sl] = jnp.where(mine, val_a[sl], 0.0)
      for p, b in enumerate(all_bins):
        d = lax.shift_right_logical(k, jnp.int32(p * 10)) & jnp.int32(NB - 1)
        cnt, lastm = plsc.scan_count(d)
        plsc.addupdate_scatter(b, [d], cnt, mask=lastm)

    def _scan(j, carry):
      sl = pl.ds(j * LN, LN)
      out = []
      for b, cp in zip(all_bins, carry):
        v = b[sl]
        inc = plsc.cumsum(v)
        b[sl] = inc - v + cp
        out.append(cp + jnp.sum(v))
      return tuple(out)

    pl.loop(0, NB // LN,
            init_carry=(jnp.int32(0), jnp.int32(0), jnp.int32(0)))(_scan)

    for p, (ksrc, vsrc, kdst, vdst) in enumerate((
        (key_a, val_a, key_b, val_b),
        (key_b, val_b, key_a, val_a),
        (key_a, val_a, key_b, val_b),
    )):
      shift = jnp.int32(p * 10)
      b = all_bins[p]

      @pl.loop(0, nvr)
      def _permute(i):
        sl = pl.ds(i * LN, LN)
        k = ksrc[sl]
        x = vsrc[sl]
        d = lax.shift_right_logical(k, shift) & jnp.int32(NB - 1)
        cnt, lastm = plsc.scan_count(d)
        pos = plsc.load_gather(b, [d]) + cnt - 1
        plsc.store_scatter(kdst, [pos], k)
        plsc.store_scatter(vdst, [pos], x)
        plsc.addupdate_scatter(b, [d], cnt, mask=lastm)

    def _suffix(i, carry):
      sl = pl.ds((nvr - 1 - i) * LN, LN)
      v = val_b[sl]
      rs = plsc.cumsum(lax.rev(v, (0,)))
      outs[sl] = lax.rev(rs, (0,)) + carry
      return carry + jnp.sum(v)

    pl.loop(0, nvr, init_carry=jnp.float32(0.0))(_suffix)

    pltpu.sync_copy(outs, out_hbm.at[pl.ds(pl.multiple_of(s * CAP, 8), CAP)])


def _post_body(suf_ref, counts_ref, inp_ref, out_ref):
  col = lax.broadcasted_iota(jnp.int32, (1, CAP), 1)
  acc = jnp.zeros((1, CAP), jnp.float32)
  for s in range(NSEG):
    row = suf_ref[pl.ds(s, 1), :]
    m = counts_ref[0, s]
    acc = acc + jnp.where(col < m, jnp.log(row + EPS), 0.0)
  total = jnp.sum(acc) - jnp.sum(inp_ref[...])
  out_ref[0, 0] = total / NSEG


_post = pl.pallas_call(
    _post_body,
    out_shape=jax.ShapeDtypeStruct((1, 1), jnp.float32),
    in_specs=(
        pl.BlockSpec(memory_space=pltpu.VMEM),
        pl.BlockSpec(memory_space=pltpu.SMEM),
        pl.BlockSpec(memory_space=pltpu.VMEM),
    ),
    out_specs=pl.BlockSpec(memory_space=pltpu.SMEM),
)


def kernel(input, target, batch):
  inp2 = input.reshape(8, N // 8)
  tgt2 = target.reshape(8, N // 8)
  bat2 = batch.astype(jnp.int32).reshape(8, N // 8)
  keys2, vals2, starts, counts = _pre(tgt2, inp2, bat2)
  suffix = _sc_segsort(
      keys2.reshape(N),
      vals2.reshape(N),
      batch.astype(jnp.int32),
      starts.reshape(NSEG),
  )
  out = _post(suffix.reshape(NSEG, CAP), counts, inp2)
  return out.reshape(())


# speedup vs baseline: 3.3311x; 1.0068x over previous
import dataclasses
import functools

import jax
import jax.numpy as jnp
from jax import lax
from jax.experimental import pallas as pl
from jax.experimental.pallas import tpu as pltpu
from jax.experimental.pallas import tpu_sc as plsc

N = 32768
NSEG = 16
CAP = 4096
NB = 1024
LN = 16
SENT = (1 << 30) - 1
EPS = 1e-12

LOG2_POLY = (0.014778720765725225, -0.0768487259661564, 0.19042083139132762,
             -0.32311593512980535, 0.4724995251905284, -0.7203866119437417,
             1.4426521110421746, 3.196978291106431e-07)
LN2 = 0.6931471805599453


def _ln(y):
  bits = lax.bitcast_convert_type(y, jnp.int32)
  e = lax.shift_right_logical(bits, jnp.int32(23)) - jnp.int32(127)
  mbits = (bits & jnp.int32(0x7FFFFF)) | jnp.int32(0x3F800000)
  r = lax.bitcast_convert_type(mbits, jnp.float32) - 1.0
  acc = jnp.full_like(r, LOG2_POLY[0])
  for co in LOG2_POLY[1:]:
    acc = acc * r + jnp.float32(co)
  return (e.astype(jnp.float32) + acc) * jnp.float32(LN2)


def _sc_compiler_params():
  cp = pltpu.CompilerParams()
  if "needs_layout_passes" in pltpu.CompilerParams.__dataclass_fields__:
    cp = dataclasses.replace(cp, needs_layout_passes=False)
  return cp


_sc_mesh = plsc.VectorSubcoreMesh(core_axis_name="c", subcore_axis_name="s")


@functools.partial(
    pl.kernel,
    out_type=jax.ShapeDtypeStruct((LN,), jnp.float32),
    mesh=_sc_mesh,
    scratch_types=[
        pltpu.VMEM((CAP,), jnp.int32),
        pltpu.VMEM((CAP,), jnp.float32),
        pltpu.VMEM((CAP,), jnp.int32),
        pltpu.VMEM((CAP,), jnp.float32),
        pltpu.VMEM((CAP,), jnp.float32),
        pltpu.VMEM((CAP,), jnp.int32),
        pltpu.VMEM((NB,), jnp.int32),
        pltpu.VMEM((NB,), jnp.int32),
        pltpu.VMEM((NB,), jnp.int32),
        pltpu.VMEM((LN,), jnp.int32),
        pltpu.VMEM((LN * LN,), jnp.int32),
        pltpu.VMEM((LN,), jnp.float32),
        pltpu.VMEM((LN * LN,), jnp.float32),
        pltpu.VMEM_SHARED((LN * LN,), jnp.int32),
        pltpu.VMEM_SHARED((LN * LN,), jnp.float32),
        pltpu.SemaphoreType.DMA,
    ],
    compiler_params=_sc_compiler_params(),
)
def _sc_loss(inp_hbm, tgt_hbm, bat_hbm, out_hbm,
             key_a, val_a, key_b, val_b, twin, bwin,
             bins0, bins1, bins2, hrow, hmat, prow, pmat,
             sh_i, sh_f, sem):
  c = lax.axis_index("c")
  s = lax.axis_index("s")

  @pl.when(c == 0)
  def _worker():
    li = lax.iota(jnp.int32, 16)
    all_bins = (bins0, bins1, bins2)

    ch = N // LN
    pltpu.sync_copy(bat_hbm.at[pl.ds(s * ch, ch)], bwin.at[pl.ds(0, ch)])
    hrow[...] = jnp.zeros((LN,), jnp.int32)

    @pl.loop(0, ch // LN)
    def _seg_hist(i):
      d = bwin[pl.ds(i * LN, LN)]
      cnt, lastm = plsc.scan_count(d)
      plsc.addupdate_scatter(hrow, [d], cnt, mask=lastm)

    pltpu.sync_copy(hrow, sh_i.at[pl.ds(pl.multiple_of(s * LN, 8), LN)])
    plsc.subcore_barrier()
    pltpu.sync_copy(sh_i, hmat)
    counts_v = jnp.zeros((LN,), jnp.int32)
    for r in range(LN):
      counts_v = counts_v + hmat[pl.ds(r * LN, LN)]
    starts_v = plsc.cumsum(counts_v) - counts_v
    my_start = jnp.sum(jnp.where(li == s, starts_v, 0))
    my_cnt = jnp.sum(jnp.where(li == s, counts_v, 0))
    seg_end = my_start + my_cnt
    base = pl.multiple_of(
        jnp.minimum(my_start & jnp.int32(~7), jnp.int32(N - CAP)), 8)
    o_end = seg_end - base
    nvr = (o_end + LN - 1) >> 4

    cp1 = pltpu.async_copy(tgt_hbm.at[pl.ds(base, CAP)], twin, sem)
    cp2 = pltpu.async_copy(inp_hbm.at[pl.ds(base, CAP)], val_a, sem)
    cp3 = pltpu.async_copy(bat_hbm.at[pl.ds(base, CAP)], bwin, sem)
    cp1.wait()
    cp2.wait()
    cp3.wait()

    @pl.loop(0, NB // LN)
    def _zero(j):
      z = jnp.zeros((LN,), jnp.int32)
      for b in all_bins:
        b[pl.ds(j * LN, LN)] = z

    def _mask_hist(i, sum_inp):
      sl = pl.ds(i * LN, LN)
      mine = bwin[sl] == s
      tb = lax.bitcast_convert_type(twin[sl], jnp.int32)
      k = jnp.where(mine, jnp.int32(0x3F7FFFFF) - tb, jnp.int32(SENT))
      key_a[sl] = k
      x = val_a[sl]
      sum_inp = sum_inp + jnp.sum(jnp.where(mine, x, 0.0))
      val_a[sl] = jnp.where(mine, jnp.exp(x), 0.0)
      for p, b in enumerate(all_bins):
        d = lax.shift_right_logical(k, jnp.int32(p * 10)) & jnp.int32(NB - 1)
        cnt, lastm = plsc.scan_count(d)
        plsc.addupdate_scatter(b, [d], cnt, mask=lastm)
      return sum_inp

    sum_inp = pl.loop(0, nvr, init_carry=jnp.float32(0.0))(_mask_hist)

    def _scan(j, carry):
      sl = pl.ds(j * LN, LN)
      out = []
      for b, cp in zip(all_bins, carry):
        v = b[sl]
        inc = plsc.cumsum(v)
        b[sl] = inc - v + cp
        out.append(cp + jnp.sum(v))
      return tuple(out)

    pl.loop(0, NB // LN,
            init_carry=(jnp.int32(0), jnp.int32(0), jnp.int32(0)))(_scan)

    for p, (ksrc, vsrc, kdst, vdst) in enumerate((
        (key_a, val_a, key_b, val_b),
        (key_b, val_b, key_a, val_a),
        (key_a, val_a, key_b, val_b),
    )):
      shift = jnp.int32(p * 10)
      b = all_bins[p]

      @pl.loop(0, nvr)
      def _permute(i):
        sl = pl.ds(i * LN, LN)
        k = ksrc[sl]
        x = vsrc[sl]
        d = lax.shift_right_logical(k, shift) & jnp.int32(NB - 1)
        cnt, lastm = plsc.scan_count(d)
        pos = plsc.load_gather(b, [d]) + cnt - 1
        plsc.store_scatter(kdst, [pos], k)
        plsc.store_scatter(vdst, [pos], x)
        plsc.addupdate_scatter(b, [d], cnt, mask=lastm)

    prow[...] = jnp.zeros((LN,), jnp.float32)

    def _suffix(i, sacc):
      j = (nvr - 1 - i) * LN
      sl = pl.ds(j, LN)
      v = val_b[sl]
      rs = plsc.cumsum(lax.rev(v, (0,)))
      suf = lax.rev(rs, (0,)) + sacc
      valid = (j + li) < my_cnt
      prow[...] = prow[...] + jnp.where(
          valid, _ln(suf + jnp.float32(EPS)), 0.0)
      return sacc + jnp.sum(v)

    pl.loop(0, nvr, init_carry=jnp.float32(0.0))(_suffix)

    partial = jnp.sum(prow[...]) - sum_inp
    prow[...] = jnp.where(li == 0, partial, 0.0)
    pltpu.sync_copy(prow, sh_f.at[pl.ds(pl.multiple_of(s * LN, 8), LN)])
    plsc.subcore_barrier()

    @pl.when(s == 0)
    def _final():
      pltpu.sync_copy(sh_f, pmat)
      facc = jnp.zeros((LN,), jnp.float32)
      for r in range(LN):
        facc = facc + pmat[pl.ds(r * LN, LN)]
      total = jnp.sum(facc) * jnp.float32(1.0 / NSEG)
      prow[...] = jnp.where(li == 0, total, 0.0)
      pltpu.sync_copy(prow, out_hbm)


def kernel(input, target, batch):
  out = _sc_loss(input, target, batch.astype(jnp.int32))
  return out[0]


# speedup vs baseline: 3.3773x; 1.0138x over previous
import dataclasses
import functools

import jax
import jax.numpy as jnp
from jax import lax
from jax.experimental import pallas as pl
from jax.experimental.pallas import tpu as pltpu
from jax.experimental.pallas import tpu_sc as plsc

N = 32768
NSEG = 16
CAP = 4096
NB = 1024
LN = 16
SENT = (1 << 30) - 1
EPS = 1e-12

LOG2_POLY = (0.014778720765725225, -0.0768487259661564, 0.19042083139132762,
             -0.32311593512980535, 0.4724995251905284, -0.7203866119437417,
             1.4426521110421746, 3.196978291106431e-07)
LN2 = 0.6931471805599453


def _ln(y):
  bits = lax.bitcast_convert_type(y, jnp.int32)
  e = lax.shift_right_logical(bits, jnp.int32(23)) - jnp.int32(127)
  mbits = (bits & jnp.int32(0x7FFFFF)) | jnp.int32(0x3F800000)
  r = lax.bitcast_convert_type(mbits, jnp.float32) - 1.0
  acc = jnp.full_like(r, LOG2_POLY[0])
  for co in LOG2_POLY[1:]:
    acc = acc * r + jnp.float32(co)
  return (e.astype(jnp.float32) + acc) * jnp.float32(LN2)


def _sc_compiler_params():
  cp = pltpu.CompilerParams()
  if "needs_layout_passes" in pltpu.CompilerParams.__dataclass_fields__:
    cp = dataclasses.replace(cp, needs_layout_passes=False)
  return cp


_sc_mesh = plsc.VectorSubcoreMesh(
    core_axis_name="c", subcore_axis_name="s", num_cores=1)


@functools.partial(
    pl.kernel,
    out_type=jax.ShapeDtypeStruct((LN,), jnp.float32),
    mesh=_sc_mesh,
    scratch_types=[
        pltpu.VMEM((CAP,), jnp.int32),
        pltpu.VMEM((CAP,), jnp.int32),
        pltpu.VMEM((CAP,), jnp.int32),
        pltpu.VMEM((CAP,), jnp.int32),
        pltpu.VMEM((CAP,), jnp.float32),
        pltpu.VMEM((CAP,), jnp.float32),
        pltpu.VMEM((CAP,), jnp.int32),
        pltpu.VMEM((NB,), jnp.int32),
        pltpu.VMEM((NB,), jnp.int32),
        pltpu.VMEM((NB,), jnp.int32),
        pltpu.VMEM((LN,), jnp.int32),
        pltpu.VMEM((LN * LN,), jnp.int32),
        pltpu.VMEM((LN,), jnp.float32),
        pltpu.VMEM((LN * LN,), jnp.float32),
        pltpu.VMEM_SHARED((LN * LN,), jnp.int32),
        pltpu.VMEM_SHARED((LN * LN,), jnp.float32),
        pltpu.SemaphoreType.DMA,
    ],
    compiler_params=_sc_compiler_params(),
)
def _sc_loss(inp_hbm, tgt_hbm, bat_hbm, out_hbm,
             key_a, key_b, idx0, idx1, val_a, twin, bwin,
             bins0, bins1, bins2, hrow, hmat, prow, pmat,
             sh_i, sh_f, sem):
  c = lax.axis_index("c")
  s = lax.axis_index("s")

  @pl.when(c == 0)
  def _worker():
    li = lax.iota(jnp.int32, 16)
    all_bins = (bins0, bins1, bins2)

    ch = N // LN
    pltpu.sync_copy(bat_hbm.at[pl.ds(s * ch, ch)], bwin.at[pl.ds(0, ch)])
    hrow[...] = jnp.zeros((LN,), jnp.int32)

    @pl.loop(0, ch // (2 * LN))
    def _seg_hist(i):
      for u in range(2):
        d = bwin[pl.ds(i * 2 * LN + u * LN, LN)]
        cnt, lastm = plsc.scan_count(d)
        plsc.addupdate_scatter(hrow, [d], cnt, mask=lastm)

    pltpu.sync_copy(hrow, sh_i.at[pl.ds(pl.multiple_of(s * LN, 8), LN)])
    plsc.subcore_barrier()
    pltpu.sync_copy(sh_i, hmat)
    counts_v = jnp.zeros((LN,), jnp.int32)
    for r in range(LN):
      counts_v = counts_v + hmat[pl.ds(r * LN, LN)]
    starts_v = plsc.cumsum(counts_v) - counts_v
    my_start = jnp.sum(jnp.where(li == s, starts_v, 0))
    my_cnt = jnp.sum(jnp.where(li == s, counts_v, 0))
    seg_end = my_start + my_cnt
    base = pl.multiple_of(
        jnp.minimum(my_start & jnp.int32(~7), jnp.int32(N - CAP)), 8)
    o_end = seg_end - base
    nv2 = (o_end + 2 * LN - 1) >> 5

    cp1 = pltpu.async_copy(tgt_hbm.at[pl.ds(base, CAP)], twin, sem)
    cp2 = pltpu.async_copy(inp_hbm.at[pl.ds(base, CAP)], val_a, sem)
    cp3 = pltpu.async_copy(bat_hbm.at[pl.ds(base, CAP)], bwin, sem)

    @pl.loop(0, NB // LN)
    def _zero(j):
      z = jnp.zeros((LN,), jnp.int32)
      for b in all_bins:
        b[pl.ds(j * LN, LN)] = z

    cp1.wait()
    cp2.wait()
    cp3.wait()

    def _mask_hist(i, carry):
      sum_inp, sum_exp = carry
      for u in range(2):
        sl = pl.ds(i * 2 * LN + u * LN, LN)
        mine = bwin[sl] == s
        tb = lax.bitcast_convert_type(twin[sl], jnp.int32)
        k = jnp.where(mine, jnp.int32(0x3F7FFFFF) - tb, jnp.int32(SENT))
        key_a[sl] = k
        x = val_a[sl]
        sum_inp = sum_inp + jnp.sum(jnp.where(mine, x, 0.0))
        e = jnp.where(mine, jnp.exp(x), 0.0)
        val_a[sl] = e
        sum_exp = sum_exp + jnp.sum(e)
        for p, b in enumerate(all_bins):
          d = lax.shift_right_logical(k, jnp.int32(p * 10)) & jnp.int32(NB - 1)
          cnt, lastm = plsc.scan_count(d)
          plsc.addupdate_scatter(b, [d], cnt, mask=lastm)
      return sum_inp, sum_exp

    sum_inp, sum_exp = pl.loop(
        0, nv2, init_carry=(jnp.float32(0.0), jnp.float32(0.0)))(_mask_hist)

    def _scan(j, carry):
      sl = pl.ds(j * LN, LN)
      out = []
      for b, cp in zip(all_bins, carry):
        v = b[sl]
        inc = plsc.cumsum(v)
        b[sl] = inc - v + cp
        out.append(cp + jnp.sum(v))
      return tuple(out)

    pl.loop(0, NB // LN,
            init_carry=(jnp.int32(0), jnp.int32(0), jnp.int32(0)))(_scan)

    @pl.loop(0, nv2)
    def _permute0(i):
      for u in range(2):
        off = i * 2 * LN + u * LN
        sl = pl.ds(off, LN)
        k = key_a[sl]
        d = k & jnp.int32(NB - 1)
        cnt, lastm = plsc.scan_count(d)
        pos = plsc.load_gather(bins0, [d]) + cnt - 1
        plsc.store_scatter(key_b, [pos], k)
        plsc.store_scatter(idx0, [pos], off + li)
        plsc.addupdate_scatter(bins0, [d], cnt, mask=lastm)

    @pl.loop(0, nv2)
    def _permute1(i):
      for u in range(2):
        sl = pl.ds(i * 2 * LN + u * LN, LN)
        k = key_b[sl]
        ix = idx0[sl]
        d = lax.shift_right_logical(k, jnp.int32(10)) & jnp.int32(NB - 1)
        cnt, lastm = plsc.scan_count(d)
        pos = plsc.load_gather(bins1, [d]) + cnt - 1
        plsc.store_scatter(key_a, [pos], k)
        plsc.store_scatter(idx1, [pos], ix)
        plsc.addupdate_scatter(bins1, [d], cnt, mask=lastm)

    @pl.loop(0, nv2)
    def _permute2(i):
      for u in range(2):
        sl = pl.ds(i * 2 * LN + u * LN, LN)
        k = key_a[sl]
        ix = idx1[sl]
        d = lax.shift_right_logical(k, jnp.int32(20)) & jnp.int32(NB - 1)
        cnt, lastm = plsc.scan_count(d)
        pos = plsc.load_gather(bins2, [d]) + cnt - 1
        plsc.store_scatter(idx0, [pos], ix)
        plsc.addupdate_scatter(bins2, [d], cnt, mask=lastm)

    prow[...] = jnp.zeros((LN,), jnp.float32)

    def _suffix(i, rem):
      for u in range(2):
        j = i * 2 * LN + u * LN
        ids = idx0[pl.ds(j, LN)]
        v = plsc.load_gather(val_a, [ids])
        pref = plsc.cumsum(v)
        suf = jnp.maximum(rem - pref + v, 0.0)
        valid = (j + li) < my_cnt
        prow[...] = prow[...] + jnp.where(
            valid, _ln(suf + jnp.float32(EPS)), 0.0)
        rem = rem - jnp.sum(v)
      return rem

    pl.loop(0, nv2, init_carry=sum_exp)(_suffix)

    partial = jnp.sum(prow[...]) - sum_inp
    prow[...] = jnp.where(li == 0, partial, 0.0)
    pltpu.sync_copy(prow, sh_f.at[pl.ds(pl.multiple_of(s * LN, 8), LN)])
    plsc.subcore_barrier()

    @pl.when(s == 0)
    def _final():
      pltpu.sync_copy(sh_f, pmat)
      facc = jnp.zeros((LN,), jnp.float32)
      for r in range(LN):
        facc = facc + pmat[pl.ds(r * LN, LN)]
      total = jnp.sum(facc) * jnp.float32(1.0 / NSEG)
      prow[...] = jnp.where(li == 0, total, 0.0)
      pltpu.sync_copy(prow, out_hbm)


def kernel(input, target, batch):
  out = _sc_loss(input, target, batch.astype(jnp.int32))
  return out[0]


# speedup vs baseline: 3.7153x; 1.1001x over previous
import dataclasses
import functools

import jax
import jax.numpy as jnp
from jax import lax
from jax.experimental import pallas as pl
from jax.experimental.pallas import tpu as pltpu
from jax.experimental.pallas import tpu_sc as plsc

N = 32768
NSEG = 16
CAP = 4096
NB = 1024
LN = 16
SENT = (1 << 30) - 1
EPS = 1e-12

LOG2_POLY = (0.043928627847900574, -0.18983244652658576, 0.4115614823104106,
             -0.7072534335743472, 1.441592077206549, 1.4390929995776245e-05)
LN2 = 0.6931471805599453


def _ln(y):
  a0, a1, a2, a3, a4, a5 = (jnp.float32(co) for co in LOG2_POLY)
  bits = lax.bitcast_convert_type(y, jnp.int32)
  e = lax.shift_right_logical(bits, jnp.int32(23)) - jnp.int32(127)
  mbits = (bits & jnp.int32(0x7FFFFF)) | jnp.int32(0x3F800000)
  r = lax.bitcast_convert_type(mbits, jnp.float32) - 1.0
  r2 = r * r
  p = ((a0 * r + a1) * r2 + (a2 * r + a3)) * r2 + (a4 * r + a5)
  return (e.astype(jnp.float32) + p) * jnp.float32(LN2)


def _sc_compiler_params():
  cp = pltpu.CompilerParams()
  if "needs_layout_passes" in pltpu.CompilerParams.__dataclass_fields__:
    cp = dataclasses.replace(cp, needs_layout_passes=False)
  return cp


_sc_mesh = plsc.VectorSubcoreMesh(
    core_axis_name="c", subcore_axis_name="s", num_cores=1)


@functools.partial(
    pl.kernel,
    out_type=jax.ShapeDtypeStruct((LN,), jnp.float32),
    mesh=_sc_mesh,
    scratch_types=[
        pltpu.VMEM((CAP,), jnp.int32),
        pltpu.VMEM((CAP,), jnp.int32),
        pltpu.VMEM((CAP,), jnp.int32),
        pltpu.VMEM((CAP,), jnp.int32),
        pltpu.VMEM((CAP,), jnp.float32),
        pltpu.VMEM((CAP,), jnp.float32),
        pltpu.VMEM((CAP,), jnp.int32),
        pltpu.VMEM((NB,), jnp.int32),
        pltpu.VMEM((NB,), jnp.int32),
        pltpu.VMEM((NB,), jnp.int32),
        pltpu.VMEM((LN,), jnp.int32),
        pltpu.VMEM((LN * LN,), jnp.int32),
        pltpu.VMEM((LN,), jnp.float32),
        pltpu.VMEM((LN * LN,), jnp.float32),
        pltpu.VMEM_SHARED((LN * LN,), jnp.int32),
        pltpu.VMEM_SHARED((LN * LN,), jnp.float32),
        pltpu.SemaphoreType.DMA,
    ],
    compiler_params=_sc_compiler_params(),
)
def _sc_loss(inp_hbm, tgt_hbm, bat_hbm, out_hbm,
             key_a, key_b, idx0, idx1, val_a, twin, bwin,
             bins0, bins1, bins2, hrow, hmat, prow, pmat,
             sh_i, sh_f, sem):
  c = lax.axis_index("c")
  s = lax.axis_index("s")

  @pl.when(c == 0)
  def _worker():
    li = lax.iota(jnp.int32, 16)
    all_bins = (bins0, bins1, bins2)

    ch = N // LN
    pltpu.sync_copy(bat_hbm.at[pl.ds(s * ch, ch)], bwin.at[pl.ds(0, ch)])
    hrow[...] = jnp.zeros((LN,), jnp.int32)

    @pl.loop(0, ch // (2 * LN))
    def _seg_hist(i):
      for u in range(2):
        d = bwin[pl.ds(i * 2 * LN + u * LN, LN)]
        cnt, lastm = plsc.scan_count(d)
        plsc.addupdate_scatter(hrow, [d], cnt, mask=lastm)

    pltpu.sync_copy(hrow, sh_i.at[pl.ds(pl.multiple_of(s * LN, 8), LN)])
    plsc.subcore_barrier()
    pltpu.sync_copy(sh_i, hmat)
    counts_v = jnp.zeros((LN,), jnp.int32)
    for r in range(LN):
      counts_v = counts_v + hmat[pl.ds(r * LN, LN)]
    starts_v = plsc.cumsum(counts_v) - counts_v
    my_start = jnp.sum(jnp.where(li == s, starts_v, 0))
    my_cnt = jnp.sum(jnp.where(li == s, counts_v, 0))
    seg_end = my_start + my_cnt
    base = pl.multiple_of(
        jnp.minimum(my_start & jnp.int32(~7), jnp.int32(N - CAP)), 8)
    o_end = seg_end - base
    nv2 = (o_end + 2 * LN - 1) >> 5

    cp1 = pltpu.async_copy(tgt_hbm.at[pl.ds(base, CAP)], twin, sem)
    cp2 = pltpu.async_copy(inp_hbm.at[pl.ds(base, CAP)], val_a, sem)
    cp3 = pltpu.async_copy(bat_hbm.at[pl.ds(base, CAP)], bwin, sem)

    @pl.loop(0, NB // LN)
    def _zero(j):
      z = jnp.zeros((LN,), jnp.int32)
      for b in all_bins:
        b[pl.ds(j * LN, LN)] = z

    cp1.wait()
    cp2.wait()
    cp3.wait()

    def _mask_hist(i, carry):
      sum_inp, sum_exp = carry
      for u in range(2):
        sl = pl.ds(i * 2 * LN + u * LN, LN)
        mine = bwin[sl] == s
        tb = lax.bitcast_convert_type(twin[sl], jnp.int32)
        k = jnp.where(mine, jnp.int32(0x3F7FFFFF) - tb, jnp.int32(SENT))
        key_a[sl] = k
        x = val_a[sl]
        sum_inp = sum_inp + jnp.sum(jnp.where(mine, x, 0.0))
        e = jnp.where(mine, jnp.exp(x), 0.0)
        val_a[sl] = e
        sum_exp = sum_exp + jnp.sum(e)
        for p, b in enumerate(all_bins):
          d = lax.shift_right_logical(k, jnp.int32(p * 10)) & jnp.int32(NB - 1)
          cnt, lastm = plsc.scan_count(d)
          plsc.addupdate_scatter(b, [d], cnt, mask=lastm)
      return sum_inp, sum_exp

    sum_inp, sum_exp = pl.loop(
        0, nv2, init_carry=(jnp.float32(0.0), jnp.float32(0.0)))(_mask_hist)

    def _scan(j, carry):
      sl = pl.ds(j * LN, LN)
      out = []
      for b, cp in zip(all_bins, carry):
        v = b[sl]
        inc = plsc.cumsum(v)
        b[sl] = inc - v + cp
        out.append(cp + jnp.sum(v))
      return tuple(out)

    pl.loop(0, NB // LN,
            init_carry=(jnp.int32(0), jnp.int32(0), jnp.int32(0)))(_scan)

    @pl.loop(0, nv2)
    def _permute0(i):
      for u in range(2):
        off = i * 2 * LN + u * LN
        sl = pl.ds(off, LN)
        k = key_a[sl]
        d = k & jnp.int32(NB - 1)
        cnt, lastm = plsc.scan_count(d)
        pos = plsc.load_gather(bins0, [d]) + cnt - 1
        plsc.store_scatter(key_b, [pos], k)
        plsc.store_scatter(idx0, [pos], off + li)
        plsc.addupdate_scatter(bins0, [d], cnt, mask=lastm)

    @pl.loop(0, nv2)
    def _permute1(i):
      for u in range(2):
        sl = pl.ds(i * 2 * LN + u * LN, LN)
        k = key_b[sl]
        ix = idx0[sl]
        d = lax.shift_right_logical(k, jnp.int32(10)) & jnp.int32(NB - 1)
        cnt, lastm = plsc.scan_count(d)
        pos = plsc.load_gather(bins1, [d]) + cnt - 1
        plsc.store_scatter(key_a, [pos], k)
        plsc.store_scatter(idx1, [pos], ix)
        plsc.addupdate_scatter(bins1, [d], cnt, mask=lastm)

    @pl.loop(0, nv2)
    def _permute2(i):
      for u in range(2):
        sl = pl.ds(i * 2 * LN + u * LN, LN)
        k = key_a[sl]
        ix = idx1[sl]
        d = lax.shift_right_logical(k, jnp.int32(20)) & jnp.int32(NB - 1)
        cnt, lastm = plsc.scan_count(d)
        pos = plsc.load_gather(bins2, [d]) + cnt - 1
        plsc.store_scatter(idx0, [pos], ix)
        plsc.addupdate_scatter(bins2, [d], cnt, mask=lastm)

    prow[...] = jnp.zeros((LN,), jnp.float32)
    pmat[pl.ds(0, LN)] = jnp.zeros((LN,), jnp.float32)

    def _suffix(i, rem):
      j0 = i * 2 * LN
      j1 = j0 + LN
      ids0 = idx0[pl.ds(j0, LN)]
      ids1 = idx0[pl.ds(j1, LN)]
      v0 = plsc.load_gather(val_a, [ids0])
      v1 = plsc.load_gather(val_a, [ids1])
      pref0 = plsc.cumsum(v0)
      pref1 = plsc.cumsum(v1)
      s0 = jnp.sum(v0)
      s1 = jnp.sum(v1)
      suf0 = jnp.maximum(rem - pref0 + v0, 0.0)
      suf1 = jnp.maximum((rem - s0) - pref1 + v1, 0.0)
      prow[...] = prow[...] + jnp.where(
          (j0 + li) < my_cnt, _ln(suf0 + jnp.float32(EPS)), 0.0)
      pmat[pl.ds(0, LN)] = pmat[pl.ds(0, LN)] + jnp.where(
          (j1 + li) < my_cnt, _ln(suf1 + jnp.float32(EPS)), 0.0)
      return rem - s0 - s1

    pl.loop(0, nv2, init_carry=sum_exp)(_suffix)
    prow[...] = prow[...] + pmat[pl.ds(0, LN)]

    partial = jnp.sum(prow[...]) - sum_inp
    prow[...] = jnp.where(li == 0, partial, 0.0)
    pltpu.sync_copy(prow, sh_f.at[pl.ds(pl.multiple_of(s * LN, 8), LN)])
    plsc.subcore_barrier()

    @pl.when(s == 0)
    def _final():
      pltpu.sync_copy(sh_f, pmat)
      facc = jnp.zeros((LN,), jnp.float32)
      for r in range(LN):
        facc = facc + pmat[pl.ds(r * LN, LN)]
      total = jnp.sum(facc) * jnp.float32(1.0 / NSEG)
      prow[...] = jnp.where(li == 0, total, 0.0)
      pltpu.sync_copy(prow, out_hbm)


def kernel(input, target, batch):
  out = _sc_loss(input, target, batch.astype(jnp.int32))
  return out[0]
